# Initial kernel scaffold; baseline (speedup 1.0000x reference)
import jax
import jax.numpy as jnp
from jax.experimental import pallas as pl


def kernel(x, edge_index, edge_type, gene_emb, W1, root1, b1, W2, root2, b2, lw1, lb1, lw2, lb2):
    raise NotImplementedError("write your pallas kernel here")


# speedup vs baseline: 1.2995x; 1.2995x over previous
import functools

import jax
import jax.numpy as jnp
from jax.experimental import pallas as pl


def _mm_kernel(a_ref, b_ref, o_ref):
    @pl.when(pl.program_id(2) == 0)
    def _zero():
        o_ref[...] = jnp.zeros_like(o_ref)

    o_ref[...] += jnp.dot(a_ref[...], b_ref[...],
                          preferred_element_type=jnp.float32)


def _ceil_to(x, m):
    return (x + m - 1) // m * m


@functools.partial(jax.jit, static_argnames=("bm", "bk", "bn"))
def _matmul(a, b, bm=512, bk=512, bn=512):
    M, K = a.shape
    K2, N = b.shape
    Mp, Kp, Np = _ceil_to(M, bm), _ceil_to(K, bk), _ceil_to(N, bn)
    a = jnp.pad(a, ((0, Mp - M), (0, Kp - K)))
    b = jnp.pad(b, ((0, Kp - K2), (0, Np - N)))
    out = pl.pallas_call(
        _mm_kernel,
        grid=(Mp // bm, Np // bn, Kp // bk),
        in_specs=[
            pl.BlockSpec((bm, bk), lambda i, j, k: (i, k)),
            pl.BlockSpec((bk, bn), lambda i, j, k: (k, j)),
        ],
        out_specs=pl.BlockSpec((bm, bn), lambda i, j, k: (i, j)),
        out_shape=jax.ShapeDtypeStruct((Mp, Np), jnp.float32),
    )(a, b)
    return out[:M, :N]


def _rgcn_layer(h, src, dst, edge_type, W, root, bias):
    N, d_in = h.shape
    R, _, d_out = W.shape
    Wf = jnp.transpose(W, (1, 0, 2)).reshape(d_in, R * d_out)
    xw = _matmul(h, Wf).reshape(N, R, d_out)
    flat_idx = src * R + edge_type
    msg = xw.reshape(N * R, d_out)[flat_idx]
    seg = dst * R + edge_type
    s = jax.ops.segment_sum(msg, seg, num_segments=N * R)
    c = jax.ops.segment_sum(jnp.ones((msg.shape[0],), jnp.float32), seg,
                            num_segments=N * R)
    mean = (s / jnp.clip(c, 1.0)[:, None]).reshape(N, R, d_out).sum(axis=1)
    out = _matmul(h, root) + bias + mean
    return out


def kernel(x, edge_index, edge_type, gene_emb, W1, root1, b1, W2, root2, b2, lw1, lb1, lw2, lb2):
    h = jnp.concatenate([x, gene_emb], axis=0)
    src = edge_index[0]
    dst = edge_index[1]
    h = jax.nn.relu(_rgcn_layer(h, src, dst, edge_type, W1, root1, b1))
    h = jax.nn.relu(_rgcn_layer(h, src, dst, edge_type, W2, root2, b2))
    h = jax.nn.relu(_matmul(h, lw1) + lb1)
    emb = h
    out = _matmul(h, lw2) + lb2
    return (jax.nn.log_softmax(out, axis=-1), emb)
